# baseline (speedup 1.0000x reference)
import functools

import jax
import jax.numpy as jnp
from jax import lax
from jax.experimental import pallas as pl
from jax.experimental.pallas import tpu as pltpu
from jax.experimental.pallas import tpu_sc as plsc

_ROWS = 16384
_SEQ = 200
_D = 128
_N = _ROWS * _SEQ
_NC = 2
_NS = 16
_NW = _NC * _NS
_PER_W = _N // _NW
_CH = 200
_NBUF = 4
_NROUND = _PER_W // (_NBUF * _CH)

_mesh = plsc.VectorSubcoreMesh(core_axis_name="c", subcore_axis_name="s")


@functools.partial(
    pl.kernel,
    out_type=jax.ShapeDtypeStruct((_N, _D), jnp.float32),
    mesh=_mesh,
    scratch_types=(
        [pltpu.VMEM((2, _D), jnp.float32)]
        + [pltpu.VMEM((_CH,), jnp.int32) for _ in range(_NBUF)]
        + [pltpu.VMEM((_CH, _D), jnp.float32) for _ in range(_NBUF)]
        + [pltpu.SemaphoreType.DMA for _ in range(2 * _NBUF)]
    ),
)
def _sc_embed(pos_hbm, w_hbm, out_hbm, w_v, *bufs):
    pos_bufs = bufs[:_NBUF]
    out_bufs = bufs[_NBUF:2 * _NBUF]
    psems = bufs[2 * _NBUF:3 * _NBUF]
    osems = bufs[3 * _NBUF:]

    wid = lax.axis_index("s") * _NC + lax.axis_index("c")
    base = wid * _PER_W

    pltpu.sync_copy(w_hbm, w_v)
    w0 = [w_v[0, pl.ds(k * 16, 16)] for k in range(8)]
    df = [w_v[1, pl.ds(k * 16, 16)] - w0[k] for k in range(8)]

    def pos_copy(c, buf, sem):
        return pltpu.make_async_copy(
            pos_hbm.at[pl.ds(base + c * _CH, _CH)], buf, sem)

    def out_copy(c, buf, sem):
        return pltpu.make_async_copy(
            buf, out_hbm.at[pl.ds(base + c * _CH, _CH)], sem)

    def compute(pos_b, out_b):
        def grp(g, carry):
            jbase = g * 16
            pv = pos_b[pl.ds(jbase, 16)].astype(jnp.float32)
            for l in range(16):
                pf = pv[l]
                for k in range(8):
                    out_b[jbase + l, pl.ds(k * 16, 16)] = w0[k] + pf * df[k]
            return carry
        lax.fori_loop(0, _CH // 16, grp, 0)

    for b in range(_NBUF):
        pos_copy(b, pos_bufs[b], psems[b]).start()

    def rnd(i, carry):
        cbase = i * _NBUF
        for b in range(_NBUF):
            c = cbase + b
            pos_copy(c, pos_bufs[b], psems[b]).wait()

            @pl.when(i > 0)
            def _():
                out_copy(c - _NBUF, out_bufs[b], osems[b]).wait()

            compute(pos_bufs[b], out_bufs[b])
            out_copy(c, out_bufs[b], osems[b]).start()

            @pl.when(i < _NROUND - 1)
            def _():
                pos_copy(c + _NBUF, pos_bufs[b], psems[b]).start()
        return carry

    lax.fori_loop(0, _NROUND, rnd, 0)
    for b in range(_NBUF):
        out_copy((_NROUND - 1) * _NBUF + b, out_bufs[b], osems[b]).wait()


def kernel(pos, seg_embd_weight):
    pos_flat = pos.astype(jnp.int32).reshape(_N)
    out = _sc_embed(pos_flat, seg_embd_weight)
    return out.reshape(_ROWS, _SEQ, _D)


# speedup vs baseline: 1.0531x; 1.0531x over previous
import functools

import jax
import jax.numpy as jnp
from jax import lax
from jax.experimental import pallas as pl
from jax.experimental.pallas import tpu as pltpu
from jax.experimental.pallas import tpu_sc as plsc

_ROWS = 16384
_SEQ = 200
_D = 128
_N = _ROWS * _SEQ
_NC = 2
_NS = 16
_NW = _NC * _NS
_PER_W = _N // _NW
_CH = 400
_NBUF = 2
_NROUND = _PER_W // (_NBUF * _CH)

_mesh = plsc.VectorSubcoreMesh(core_axis_name="c", subcore_axis_name="s")


@functools.partial(
    pl.kernel,
    out_type=jax.ShapeDtypeStruct((_N, _D), jnp.float32),
    mesh=_mesh,
    scratch_types=(
        [pltpu.VMEM((2, _D), jnp.float32)]
        + [pltpu.VMEM((_CH,), jnp.int32) for _ in range(_NBUF)]
        + [pltpu.VMEM((_CH, _D), jnp.float32) for _ in range(_NBUF)]
        + [pltpu.SemaphoreType.DMA for _ in range(3 * _NBUF)]
    ),
)
def _sc_embed(pos_hbm, w_hbm, out_hbm, w_v, *bufs):
    pos_bufs = bufs[:_NBUF]
    out_bufs = bufs[_NBUF:2 * _NBUF]
    psems = bufs[2 * _NBUF:3 * _NBUF]
    osems = bufs[3 * _NBUF:]

    wid = lax.axis_index("s") * _NC + lax.axis_index("c")
    base = wid * _PER_W

    pltpu.sync_copy(w_hbm, w_v)
    w0 = [w_v[0, pl.ds(k * 16, 16)] for k in range(8)]
    df = [w_v[1, pl.ds(k * 16, 16)] - w0[k] for k in range(8)]

    def pos_copy(c, buf, sem):
        return pltpu.make_async_copy(
            pos_hbm.at[pl.ds(base + c * _CH, _CH)], buf, sem)

    _HH = _CH // 2

    def out_copy(c, buf, sem, h):
        return pltpu.make_async_copy(
            buf.at[pl.ds(h * _HH, _HH)],
            out_hbm.at[pl.ds(base + c * _CH + h * _HH, _HH)], sem)

    def compute(pos_b, out_b):
        def grp(g, carry):
            jbase = g * 16
            pv = pos_b[pl.ds(jbase, 16)].astype(jnp.float32)
            for l in range(16):
                pf = pv[l]
                for k in range(8):
                    out_b[jbase + l, pl.ds(k * 16, 16)] = w0[k] + pf * df[k]
            return carry
        lax.fori_loop(0, _CH // 16, grp, 0)

    for b in range(_NBUF):
        pos_copy(b, pos_bufs[b], psems[b]).start()

    def rnd(i, carry):
        cbase = i * _NBUF
        for b in range(_NBUF):
            c = cbase + b
            pos_copy(c, pos_bufs[b], psems[b]).wait()

            @pl.when(i > 0)
            def _():
                for h in range(2):
                    out_copy(c - _NBUF, out_bufs[b], osems[2 * b + h], h).wait()

            compute(pos_bufs[b], out_bufs[b])
            for h in range(2):
                out_copy(c, out_bufs[b], osems[2 * b + h], h).start()

            @pl.when(i < _NROUND - 1)
            def _():
                pos_copy(c + _NBUF, pos_bufs[b], psems[b]).start()
        return carry

    lax.fori_loop(0, _NROUND, rnd, 0)
    for b in range(_NBUF):
        for h in range(2):
            out_copy((_NROUND - 1) * _NBUF + b, out_bufs[b],
                     osems[2 * b + h], h).wait()


def kernel(pos, seg_embd_weight):
    pos_flat = pos.astype(jnp.int32).reshape(_N)
    out = _sc_embed(pos_flat, seg_embd_weight)
    return out.reshape(_ROWS, _SEQ, _D)
